# baseline (speedup 1.0000x reference)
import functools

import jax
import jax.numpy as jnp
from jax import lax
from jax.experimental import pallas as pl
from jax.experimental.pallas import tpu as pltpu
from jax.experimental.pallas import tpu_sc as plsc

_EMBED = 179
_OUT_D = _EMBED + 1
_TBL_ROWS = 128
_ZERO_ROW = _TBL_ROWS - 1
_TBL_FLAT = _TBL_ROWS * _OUT_D

_NC, _NS, _L = 2, 16, 16
_NW = _NC * _NS

_CHUNK = 256


def _sc_body(n_tok, idx_hbm, typ_hbm, tbl_hbm, out_hbm,
             idx0, idx1, typ0, typ1, effb_v, last_v, tbl_v,
             raw0, raw1, ts0, ts1, os0, os1):
    wid = lax.axis_index("s") * _NC + lax.axis_index("c")
    per_w = n_tok // _NW
    tile_base = wid * per_w
    n_chunks = per_w // _CHUNK
    n2 = n_chunks // 2

    iota = lax.iota(jnp.int32, _L)

    def col16(c):
        return jnp.full((_L,), c, jnp.int32)

    def idx_src(g):
        return idx_hbm.at[pl.ds(tile_base + g * _CHUNK, _CHUNK)]

    def typ_src(g):
        return typ_hbm.at[pl.ds(tile_base + g * _CHUNK, _CHUNK)]

    def out_dst(g):
        return out_hbm.at[pl.ds(tile_base + g * _CHUNK, _CHUNK)]

    def parse(idx_v, typ_v):
        for k in range(_CHUNK // _L):
            i16 = idx_v[pl.ds(k * _L, _L)]
            t16 = typ_v[pl.ds(k * _L, _L)]
            is_b = t16 == 0
            eff16 = jnp.where(is_b, i16, _ZERO_ROW)
            effb_v[pl.ds(k * _L, _L)] = eff16 * _OUT_D
            last_v[pl.ds(k * _L, _L)] = jnp.where(
                is_b, jnp.float32(-1.0), i16.astype(jnp.float32))

    def distribute(raw_v):
        @plsc.parallel_loop(0, _CHUNK // _L, unroll=2)
        def group(k):
            effb16 = effb_v[pl.ds(k * _L, _L)]
            row16 = iota + (k * _L)
            for c in range(_EMBED):
                a = plsc.load_gather(tbl_v, [effb16 + c])
                plsc.store_scatter(raw_v, [row16, col16(c)], a)
            plsc.store_scatter(raw_v, [row16, col16(_EMBED)],
                               last_v[pl.ds(k * _L, _L)])

    def tok_start(g, idx_v, typ_v, sem):
        pltpu.async_copy(idx_src(g), idx_v, sem)
        pltpu.async_copy(typ_src(g), typ_v, sem)

    def tok_wait(g, idx_v, typ_v, sem):
        pltpu.make_async_copy(idx_src(g), idx_v, sem).wait()
        pltpu.make_async_copy(typ_src(g), typ_v, sem).wait()

    def out_start(g, raw_v, sem):
        pltpu.async_copy(raw_v, out_dst(g), sem)

    def out_wait(g, raw_v, sem):
        pltpu.make_async_copy(raw_v, out_dst(g), sem).wait()

    pltpu.sync_copy(tbl_hbm, tbl_v)
    pltpu.sync_copy(idx_src(0), idx0)
    pltpu.sync_copy(typ_src(0), typ0)

    def body(gg, carry):
        g = 2 * gg
        tok_start(g + 1, idx1, typ1, ts1)
        parse(idx0, typ0)

        @pl.when(gg > 0)
        def _():
            out_wait(g - 2, raw0, os0)
        distribute(raw0)
        out_start(g, raw0, os0)

        @pl.when(gg < n2 - 1)
        def _():
            tok_start(g + 2, idx0, typ0, ts0)
        tok_wait(g + 1, idx1, typ1, ts1)
        parse(idx1, typ1)

        @pl.when(gg > 0)
        def _():
            out_wait(g - 1, raw1, os1)
        distribute(raw1)
        out_start(g + 1, raw1, os1)

        @pl.when(gg < n2 - 1)
        def _():
            tok_wait(g + 2, idx0, typ0, ts0)

        return carry

    lax.fori_loop(0, n2, body, 0)

    out_wait(n_chunks - 2, raw0, os0)
    out_wait(n_chunks - 1, raw1, os1)


@functools.partial(jax.jit, static_argnames=("n_tok",))
def _run(idx, typ, tbl_flat, n_tok):
    mesh = plsc.VectorSubcoreMesh(core_axis_name="c", subcore_axis_name="s")
    out = pl.kernel(
        functools.partial(_sc_body, n_tok),
        out_type=jax.ShapeDtypeStruct((n_tok, _OUT_D), jnp.float32),
        mesh=mesh,
        compiler_params=pltpu.CompilerParams(
            needs_layout_passes=False, use_tc_tiling_on_sc=False),
        scratch_types=[
            pltpu.VMEM((_CHUNK,), jnp.int32),
            pltpu.VMEM((_CHUNK,), jnp.int32),
            pltpu.VMEM((_CHUNK,), jnp.int32),
            pltpu.VMEM((_CHUNK,), jnp.int32),
            pltpu.VMEM((_CHUNK,), jnp.int32),
            pltpu.VMEM((_CHUNK,), jnp.float32),
            pltpu.VMEM((_TBL_FLAT,), jnp.float32),
            pltpu.VMEM((_CHUNK, _OUT_D), jnp.float32),
            pltpu.VMEM((_CHUNK, _OUT_D), jnp.float32),
            pltpu.SemaphoreType.DMA,
            pltpu.SemaphoreType.DMA,
            pltpu.SemaphoreType.DMA,
            pltpu.SemaphoreType.DMA,
        ],
    )(idx, typ, tbl_flat)
    return out


def kernel(tokens, builtin_table, variable_table):
    del variable_table
    n_tok = tokens.shape[0]
    tbl = jnp.zeros((_TBL_ROWS, _OUT_D), jnp.float32)
    tbl = tbl.at[: builtin_table.shape[0], :_EMBED].set(builtin_table)
    tbl = tbl.at[: builtin_table.shape[0], _EMBED].set(-1.0)
    idx = jnp.sum(tokens * jnp.array([[1, 0]], jnp.int32), axis=1)
    typ = jnp.sum(tokens * jnp.array([[0, 1]], jnp.int32), axis=1)
    return _run(idx, typ, tbl.reshape(_TBL_FLAT), n_tok)


# speedup vs baseline: 1.0505x; 1.0505x over previous
import functools

import jax
import jax.numpy as jnp
from jax import lax
from jax.experimental import pallas as pl
from jax.experimental.pallas import tpu as pltpu
from jax.experimental.pallas import tpu_sc as plsc

_EMBED = 179
_OUT_D = _EMBED + 1
_TBL_ROWS = 128
_ZERO_ROW = _TBL_ROWS - 1
_TBL_FLAT = _TBL_ROWS * _OUT_D

_NC, _NS, _L = 2, 16, 16
_NW = _NC * _NS

_CHUNK = 256


def _sc_body(n_tok, idx_hbm, typ_hbm, tbl_hbm, out_hbm,
             idx0, idx1, typ0, typ1, effb_v, last_v, tbl_v,
             raw0, raw1, ts0, ts1, os0, os1):
    wid = lax.axis_index("s") * _NC + lax.axis_index("c")
    per_w = n_tok // _NW
    tile_base = wid * per_w
    n_chunks = per_w // _CHUNK
    n2 = n_chunks // 2

    iota = lax.iota(jnp.int32, _L)

    def col16(c):
        return jnp.full((_L,), c, jnp.int32)

    def idx_src(g):
        return idx_hbm.at[pl.ds(tile_base + g * _CHUNK, _CHUNK)]

    def typ_src(g):
        return typ_hbm.at[pl.ds(tile_base + g * _CHUNK, _CHUNK)]

    def out_dst(g):
        return out_hbm.at[pl.ds((tile_base + g * _CHUNK) * _OUT_D // 128,
                                _CHUNK * _OUT_D // 128)]

    def parse(idx_v, typ_v):
        for k in range(_CHUNK // _L):
            i16 = idx_v[pl.ds(k * _L, _L)]
            t16 = typ_v[pl.ds(k * _L, _L)]
            is_b = t16 == 0
            eff16 = jnp.where(is_b, i16, _ZERO_ROW)
            effb_v[pl.ds(k * _L, _L)] = eff16 * _OUT_D
            last_v[pl.ds(k * _L, _L)] = jnp.where(
                is_b, jnp.float32(-1.0), i16.astype(jnp.float32))

    def distribute(raw_v):
        @plsc.parallel_loop(0, _CHUNK // _L, unroll=2)
        def group(k):
            effb16 = effb_v[pl.ds(k * _L, _L)]
            rowf16 = (iota + (k * _L)) * _OUT_D
            for c in range(_OUT_D):
                w16 = rowf16 + c
                if c == _EMBED:
                    a = last_v[pl.ds(k * _L, _L)]
                else:
                    a = plsc.load_gather(tbl_v, [effb16 + c])
                plsc.store_scatter(
                    raw_v,
                    [lax.shift_right_logical(w16, 7),
                     lax.bitwise_and(w16, 127)], a)

    def tok_start(g, idx_v, typ_v, sem):
        pltpu.async_copy(idx_src(g), idx_v, sem)
        pltpu.async_copy(typ_src(g), typ_v, sem)

    def tok_wait(g, idx_v, typ_v, sem):
        pltpu.make_async_copy(idx_src(g), idx_v, sem).wait()
        pltpu.make_async_copy(typ_src(g), typ_v, sem).wait()

    def out_start(g, raw_v, sem):
        pltpu.async_copy(raw_v, out_dst(g), sem)

    def out_wait(g, raw_v, sem):
        pltpu.make_async_copy(raw_v, out_dst(g), sem).wait()

    pltpu.sync_copy(tbl_hbm, tbl_v)
    pltpu.sync_copy(idx_src(0), idx0)
    pltpu.sync_copy(typ_src(0), typ0)

    def body(gg, carry):
        g = 2 * gg
        tok_start(g + 1, idx1, typ1, ts1)
        parse(idx0, typ0)

        @pl.when(gg > 0)
        def _():
            out_wait(g - 2, raw0, os0)
        distribute(raw0)
        out_start(g, raw0, os0)

        @pl.when(gg < n2 - 1)
        def _():
            tok_start(g + 2, idx0, typ0, ts0)
        tok_wait(g + 1, idx1, typ1, ts1)
        parse(idx1, typ1)

        @pl.when(gg > 0)
        def _():
            out_wait(g - 1, raw1, os1)
        distribute(raw1)
        out_start(g + 1, raw1, os1)

        @pl.when(gg < n2 - 1)
        def _():
            tok_wait(g + 2, idx0, typ0, ts0)

        return carry

    lax.fori_loop(0, n2, body, 0)

    out_wait(n_chunks - 2, raw0, os0)
    out_wait(n_chunks - 1, raw1, os1)


@functools.partial(jax.jit, static_argnames=("n_tok",))
def _run(idx, typ, tbl_flat, n_tok):
    mesh = plsc.VectorSubcoreMesh(core_axis_name="c", subcore_axis_name="s")
    out = pl.kernel(
        functools.partial(_sc_body, n_tok),
        out_type=jax.ShapeDtypeStruct((n_tok * _OUT_D // 128, 128),
                                      jnp.float32),
        mesh=mesh,
        compiler_params=pltpu.CompilerParams(
            needs_layout_passes=False, use_tc_tiling_on_sc=False),
        scratch_types=[
            pltpu.VMEM((_CHUNK,), jnp.int32),
            pltpu.VMEM((_CHUNK,), jnp.int32),
            pltpu.VMEM((_CHUNK,), jnp.int32),
            pltpu.VMEM((_CHUNK,), jnp.int32),
            pltpu.VMEM((_CHUNK,), jnp.int32),
            pltpu.VMEM((_CHUNK,), jnp.float32),
            pltpu.VMEM((_TBL_FLAT,), jnp.float32),
            pltpu.VMEM((_CHUNK * _OUT_D // 128, 128), jnp.float32),
            pltpu.VMEM((_CHUNK * _OUT_D // 128, 128), jnp.float32),
            pltpu.SemaphoreType.DMA,
            pltpu.SemaphoreType.DMA,
            pltpu.SemaphoreType.DMA,
            pltpu.SemaphoreType.DMA,
        ],
    )(idx, typ, tbl_flat)
    return out.reshape(n_tok, _OUT_D)


def kernel(tokens, builtin_table, variable_table):
    del variable_table
    n_tok = tokens.shape[0]
    tbl = jnp.zeros((_TBL_ROWS, _OUT_D), jnp.float32)
    tbl = tbl.at[: builtin_table.shape[0], :_EMBED].set(builtin_table)
    tbl = tbl.at[: builtin_table.shape[0], _EMBED].set(-1.0)
    idx = jnp.sum(tokens * jnp.array([[1, 0]], jnp.int32), axis=1)
    typ = jnp.sum(tokens * jnp.array([[0, 1]], jnp.int32), axis=1)
    return _run(idx, typ, tbl.reshape(_TBL_FLAT), n_tok)
